# baseline (speedup 1.0000x reference)
import jax
import jax.numpy as jnp
from jax.experimental import pallas as pl
from jax.experimental.pallas import tpu as pltpu

K = 64


def _encode_kernel(x_ref, w_ref, b_ref, out_ref):
    out_ref[...] = (
        jax.lax.dot_general(
            x_ref[...], w_ref[...],
            dimension_numbers=(((1,), (1,)), ((), ())),
            preferred_element_type=jnp.float32,
        )
        + b_ref[...]
    )


_STACK = 8
_LN = 128
_RG = 8
_CS = 4


def _topk_kernel(pre_ref, sparse_ref, idx_ref, work_ref):
    BT, CH, LN = pre_ref.shape
    H = CH * LN
    big = jnp.int32(2**30)
    k_iota = jax.lax.broadcasted_iota(jnp.int32, (BT, K), 1)

    svals = [[] for _ in range(_STACK)]
    sidx = [[] for _ in range(_STACK)]
    dmax_parts = []
    for g in range(BT // _RG):
        lane_idx = jax.lax.broadcasted_iota(jnp.int32, (_RG, LN), 1)
        init = (
            tuple(jnp.full((_RG, LN), -jnp.inf, jnp.float32)
                  for _ in range(_STACK)),
            tuple(jnp.zeros((_RG, LN), jnp.int32) for _ in range(_STACK)),
            jnp.full((_RG, LN), -jnp.inf, jnp.float32),
        )

        def body(ci, carry, g=g, lane_idx=lane_idx):
            vs, isx, d = carry
            blk = pre_ref[pl.ds(g * _RG, _RG), pl.ds(ci * _CS, _CS), :]
            vs, isx = list(vs), list(isx)
            for j in range(_CS):
                cur_v = blk[:, j, :]
                cur_i = (ci * _CS + j) * LN + lane_idx
                for r in range(_STACK):
                    sel = cur_v > vs[r]
                    hi_v = jnp.where(sel, cur_v, vs[r])
                    lo_v = jnp.where(sel, vs[r], cur_v)
                    hi_i = jnp.where(sel, cur_i, isx[r])
                    lo_i = jnp.where(sel, isx[r], cur_i)
                    vs[r], isx[r] = hi_v, hi_i
                    cur_v, cur_i = lo_v, lo_i
                d = jnp.maximum(d, cur_v)
            return tuple(vs), tuple(isx), d

        vs_g, is_g, d_g = jax.lax.fori_loop(0, CH // _CS, body, init)
        for r in range(_STACK):
            svals[r].append(vs_g[r])
            sidx[r].append(is_g[r])
        dmax_parts.append(d_g)

    sv = [jnp.concatenate(p, axis=0) for p in svals]
    si = [jnp.concatenate(p, axis=0) for p in sidx]
    dmax = jnp.concatenate(dmax_parts, axis=0)

    m_rem = jnp.max(dmax, axis=1, keepdims=True)
    cnt = jnp.zeros((BT, 1), jnp.int32)
    for r in range(_STACK):
        cnt = cnt + jnp.sum((sv[r] > m_rem).astype(jnp.int32), axis=1,
                            keepdims=True)
    ok = jnp.all(cnt >= K)

    @pl.when(ok)
    def _fast():
        def body(k, carry):
            f, fi, ptr, idxs, _ = carry
            m = jnp.max(f, axis=1, keepdims=True)
            ii = jnp.min(jnp.where(f == m, fi, big), axis=1, keepdims=True)
            adv = (f == m) & (fi == ii)
            ptr2 = ptr + adv.astype(jnp.int32)
            nv = jnp.full_like(f, -jnp.inf)
            ni = jnp.zeros_like(fi)
            for r in range(1, _STACK):
                hit = ptr2 == r
                nv = jnp.where(hit, sv[r], nv)
                ni = jnp.where(hit, si[r], ni)
            f2 = jnp.where(adv, nv, f)
            fi2 = jnp.where(adv, ni, fi)
            idxs2 = jnp.where(k_iota == k, ii, idxs)
            return f2, fi2, ptr2, idxs2, m

        _, _, _, idxs, v64 = jax.lax.fori_loop(
            0, K, body,
            (sv[0], si[0], jnp.zeros((BT, LN), jnp.int32),
             jnp.zeros((BT, K), jnp.int32), jnp.zeros((BT, 1), jnp.float32)),
        )
        idx_ref[...] = idxs
        a3 = pre_ref[...]
        v64b = v64.reshape(BT, 1, 1)
        sparse_ref[...] = jnp.where((a3 >= v64b) & (a3 > 0.0), a3, 0.0)

    @pl.when(jnp.logical_not(ok))
    def _slow():
        a = pre_ref[...].reshape(BT, H)
        work_ref[...] = a
        col = jax.lax.broadcasted_iota(jnp.int32, (BT, H), 1)

        def body(k, idxs):
            w = work_ref[...]
            m = jnp.max(w, axis=1, keepdims=True)
            amax = jnp.min(jnp.where(w == m, col, big), axis=1, keepdims=True)
            work_ref[...] = jnp.where(col == amax, -jnp.inf, w)
            return jnp.where(k_iota == k, amax, idxs)

        idxs = jax.lax.fori_loop(0, K, body, jnp.zeros((BT, K), jnp.int32))
        idx_ref[...] = idxs
        selected = work_ref[...] != a
        sparse_ref[...] = jnp.where(
            selected & (a > 0.0), a, 0.0
        ).reshape(BT, CH, LN)


def _decode_kernel(s_ref, w_ref, b_ref, out_ref):
    h = pl.program_id(1)

    @pl.when(h == 0)
    def _():
        out_ref[...] = jnp.broadcast_to(b_ref[...], out_ref.shape)

    out_ref[...] += jax.lax.dot_general(
        s_ref[...], w_ref[...],
        dimension_numbers=(((1,), (1,)), ((), ())),
        preferred_element_type=jnp.float32,
    )


def kernel(x, W_enc, b_enc, W_dec, b_dec):
    NT, D = x.shape
    H = W_enc.shape[0]
    b_enc2 = b_enc.reshape(1, H)
    b_dec2 = b_dec.reshape(1, D)

    BH = 1024
    pre_act = pl.pallas_call(
        _encode_kernel,
        grid=(H // BH,),
        in_specs=[
            pl.BlockSpec((NT, D), lambda h: (0, 0)),
            pl.BlockSpec((BH, D), lambda h: (h, 0)),
            pl.BlockSpec((1, BH), lambda h: (0, h)),
        ],
        out_specs=pl.BlockSpec((NT, BH), lambda h: (0, h)),
        out_shape=jax.ShapeDtypeStruct((NT, H), jnp.float32),
        compiler_params=pltpu.CompilerParams(
            dimension_semantics=("parallel",)
        ),
    )(x, W_enc, b_enc2)

    BT = min(64, NT)
    CH = H // _LN
    pre3 = pre_act.reshape(NT, CH, _LN)
    sparse3, topk_idx = pl.pallas_call(
        _topk_kernel,
        grid=(NT // BT,),
        in_specs=[pl.BlockSpec((BT, CH, _LN), lambda i: (i, 0, 0))],
        out_specs=[
            pl.BlockSpec((BT, CH, _LN), lambda i: (i, 0, 0)),
            pl.BlockSpec((BT, K), lambda i: (i, 0)),
        ],
        out_shape=[
            jax.ShapeDtypeStruct((NT, CH, _LN), jnp.float32),
            jax.ShapeDtypeStruct((NT, K), jnp.int32),
        ],
        scratch_shapes=[
            pltpu.VMEM((BT, H), jnp.float32),
        ],
        compiler_params=pltpu.CompilerParams(
            dimension_semantics=("parallel",)
        ),
    )(pre3)
    sparse_act = sparse3.reshape(NT, H)

    BHD = 1024
    BTD = NT // 2
    x_recon = pl.pallas_call(
        _decode_kernel,
        grid=(NT // BTD, H // BHD),
        in_specs=[
            pl.BlockSpec((BTD, BHD), lambda t, h: (t, h)),
            pl.BlockSpec((D, BHD), lambda t, h: (0, h)),
            pl.BlockSpec((1, D), lambda t, h: (0, 0)),
        ],
        out_specs=pl.BlockSpec((BTD, D), lambda t, h: (t, 0)),
        out_shape=jax.ShapeDtypeStruct((NT, D), jnp.float32),
        compiler_params=pltpu.CompilerParams(
            dimension_semantics=("parallel", "arbitrary")
        ),
    )(sparse_act, W_dec, b_dec2)

    return (x_recon, sparse_act, topk_idx)


# speedup vs baseline: 1.7729x; 1.7729x over previous
import jax
import jax.numpy as jnp
from jax.experimental import pallas as pl
from jax.experimental.pallas import tpu as pltpu

K = 64


def _encode_kernel(x_ref, w_ref, b_ref, out_ref):
    out_ref[...] = (
        jax.lax.dot_general(
            x_ref[...], w_ref[...],
            dimension_numbers=(((1,), (1,)), ((), ())),
            preferred_element_type=jnp.float32,
        )
        + b_ref[...]
    )


_ROUNDS = 8
_LN = 128


def _topk_kernel(pre_ref, sparse_ref, idx_ref, work_ref):
    BT, H = pre_ref.shape
    CH = H // _LN
    a = pre_ref[...]
    work_ref[...] = a
    k_iota = jax.lax.broadcasted_iota(jnp.int32, (BT, K), 1)
    lane_i = jax.lax.broadcasted_iota(jnp.int32, (BT, _LN), 1)
    chunk_i3 = jax.lax.broadcasted_iota(jnp.int32, (BT, CH, _LN), 1)
    big = jnp.int32(2**30)

    cand_v, cand_i = [], []
    for _ in range(_ROUNDS):
        w3 = work_ref[...].reshape(BT, CH, _LN)
        lm = jnp.max(w3, axis=1)
        eq = w3 == lm[:, None, :]
        csel = jnp.min(jnp.where(eq, chunk_i3, big), axis=1)
        cand_v.append(lm)
        cand_i.append(csel * _LN + lane_i)
        work_ref[...] = jnp.where(eq, -jnp.inf, w3).reshape(BT, H)

    def fbody(k, carry):
        f, fi, ptr, idxs, _ = carry
        m = jnp.max(f, axis=1, keepdims=True)
        ii = jnp.min(jnp.where(f == m, fi, big), axis=1, keepdims=True)
        adv = (f == m) & (fi == ii)
        ptr2 = ptr + adv.astype(jnp.int32)
        nv = jnp.full_like(f, -jnp.inf)
        ni = jnp.zeros_like(fi)
        for r in range(1, _ROUNDS):
            hit = ptr2 == r
            nv = jnp.where(hit, cand_v[r], nv)
            ni = jnp.where(hit, cand_i[r], ni)
        f2 = jnp.where(adv, nv, f)
        fi2 = jnp.where(adv, ni, fi)
        idxs2 = jnp.where(k_iota == k, ii, idxs)
        return f2, fi2, ptr2, idxs2, m

    _, _, _, idxs, v64 = jax.lax.fori_loop(
        0, K, fbody,
        (cand_v[0], cand_i[0], jnp.zeros((BT, _LN), jnp.int32),
         jnp.zeros((BT, K), jnp.int32), jnp.zeros((BT, 1), jnp.float32)),
    )
    idx_ref[...] = idxs
    ge = a >= v64
    sparse_ref[...] = jnp.where(ge & (a > 0.0), a, 0.0)

    cnt = jnp.sum(ge.astype(jnp.int32), axis=1)
    ok = jnp.all(cnt == K)

    @pl.when(jnp.logical_not(ok))
    def _slow():
        work_ref[...] = a
        col = jax.lax.broadcasted_iota(jnp.int32, (BT, H), 1)

        def body(k, idxs):
            w = work_ref[...]
            m = jnp.max(w, axis=1, keepdims=True)
            amax = jnp.min(jnp.where(w == m, col, big), axis=1, keepdims=True)
            work_ref[...] = jnp.where(col == amax, -jnp.inf, w)
            return jnp.where(k_iota == k, amax, idxs)

        idxs = jax.lax.fori_loop(0, K, body, jnp.zeros((BT, K), jnp.int32))
        idx_ref[...] = idxs
        selected = work_ref[...] != a
        sparse_ref[...] = jnp.where(selected & (a > 0.0), a, 0.0)


def _decode_kernel(s_ref, w_ref, b_ref, out_ref):
    h = pl.program_id(1)

    @pl.when(h == 0)
    def _():
        out_ref[...] = jnp.broadcast_to(b_ref[...], out_ref.shape)

    out_ref[...] += jax.lax.dot_general(
        s_ref[...], w_ref[...],
        dimension_numbers=(((1,), (1,)), ((), ())),
        preferred_element_type=jnp.float32,
    )


def kernel(x, W_enc, b_enc, W_dec, b_dec):
    NT, D = x.shape
    H = W_enc.shape[0]
    b_enc2 = b_enc.reshape(1, H)
    b_dec2 = b_dec.reshape(1, D)

    BH = 1024
    pre_act = pl.pallas_call(
        _encode_kernel,
        grid=(H // BH,),
        in_specs=[
            pl.BlockSpec((NT, D), lambda h: (0, 0)),
            pl.BlockSpec((BH, D), lambda h: (h, 0)),
            pl.BlockSpec((1, BH), lambda h: (0, h)),
        ],
        out_specs=pl.BlockSpec((NT, BH), lambda h: (0, h)),
        out_shape=jax.ShapeDtypeStruct((NT, H), jnp.float32),
        compiler_params=pltpu.CompilerParams(
            dimension_semantics=("parallel",)
        ),
    )(x, W_enc, b_enc2)

    BT = min(32, NT)
    sparse_act, topk_idx = pl.pallas_call(
        _topk_kernel,
        grid=(NT // BT,),
        in_specs=[pl.BlockSpec((BT, H), lambda i: (i, 0))],
        out_specs=[
            pl.BlockSpec((BT, H), lambda i: (i, 0)),
            pl.BlockSpec((BT, K), lambda i: (i, 0)),
        ],
        out_shape=[
            jax.ShapeDtypeStruct((NT, H), jnp.float32),
            jax.ShapeDtypeStruct((NT, K), jnp.int32),
        ],
        scratch_shapes=[
            pltpu.VMEM((BT, H), jnp.float32),
        ],
        compiler_params=pltpu.CompilerParams(
            dimension_semantics=("parallel",)
        ),
    )(pre_act)

    BHD = 1024
    BTD = NT // 2
    x_recon = pl.pallas_call(
        _decode_kernel,
        grid=(NT // BTD, H // BHD),
        in_specs=[
            pl.BlockSpec((BTD, BHD), lambda t, h: (t, h)),
            pl.BlockSpec((D, BHD), lambda t, h: (0, h)),
            pl.BlockSpec((1, D), lambda t, h: (0, 0)),
        ],
        out_specs=pl.BlockSpec((BTD, D), lambda t, h: (t, 0)),
        out_shape=jax.ShapeDtypeStruct((NT, D), jnp.float32),
        compiler_params=pltpu.CompilerParams(
            dimension_semantics=("parallel", "arbitrary")
        ),
    )(sparse_act, W_dec, b_dec2)

    return (x_recon, sparse_act, topk_idx)
